# baseline (speedup 1.0000x reference)
import functools

import jax
import jax.numpy as jnp
from jax import lax
from jax.experimental import pallas as pl
from jax.experimental.pallas import tpu as pltpu
from jax.experimental.pallas import tpu_sc as plsc

NS = 26
ND = 13
VOCAB = 100000
ED = 16
OUT_D = NS * ED + 1
LANES = 16
R = 128

_V = NS * VOCAB
_BLK = 1024
_NBLK = (_V + _BLK - 1) // _BLK
_OROWS = _BLK // 8
_VPAD = (_NBLK * _BLK) // 8
_NW = 32
_NSLOT = 3
_TPW = _NSLOT * ((_NBLK + _NSLOT * _NW - 1) // (_NSLOT * _NW))

_detile_mesh = plsc.VectorSubcoreMesh(core_axis_name="c", subcore_axis_name="s")


@functools.partial(
    pl.kernel,
    mesh=_detile_mesh,
    out_type=jax.ShapeDtypeStruct((_VPAD, 128), jnp.float32),
    scratch_types=[
        pltpu.VMEM((_NSLOT * (_BLK // 128) * ED, 128), jnp.float32),
        pltpu.VMEM((_NSLOT, _OROWS, 128), jnp.float32),
        pltpu.SemaphoreType.DMA,
        pltpu.SemaphoreType.DMA,
        pltpu.SemaphoreType.DMA,
        pltpu.SemaphoreType.DMA,
        pltpu.SemaphoreType.DMA,
        pltpu.SemaphoreType.DMA,
    ],
    compiler_params=pltpu.CompilerParams(needs_layout_passes=False),
)
def _sc_detile(tin, tout, tbuf, obuf, si0, si1, si2, so0, so1, so2):
    nc = 2
    wid = lax.axis_index("s") * nc + lax.axis_index("c")
    iota = jnp.arange(LANES, dtype=jnp.int32)
    sems_i = (si0, si1, si2)
    sems_o = (so0, so1, so2)
    last = _V + 64 - _BLK

    def base_of(t):
        return pl.multiple_of(jnp.minimum((t * _NW + wid) * _BLK, last), 128)

    def in_copies(t, slot):
        return [
            pltpu.make_async_copy(
                tin.at[pl.ds(8 * et, 8), pl.ds(base_of(t) + 128 * k, 128)],
                tbuf.at[pl.ds((slot * (_BLK // 128) + k) * ED + 8 * et, 8), :],
                sems_i[slot])
            for k in range(_BLK // 128)
            for et in range(2)
        ]

    def out_copy(t, slot):
        return pltpu.make_async_copy(
            obuf.at[slot],
            tout.at[pl.ds(pl.multiple_of(base_of(t) // 8, 16), _OROWS), :],
            sems_o[slot])

    def valid(t):
        return (t * _NW + wid) * _BLK < _V

    for s in range(_NSLOT):
        for cp in in_copies(s, s):
            cp.start()

    def phase(t, slot):
        @pl.when(valid(t))
        def _():
            @pl.when(t >= _NSLOT)
            def _():
                out_copy(t - _NSLOT, slot).wait()
            for cp in in_copies(t, slot):
                cp.wait()
            for cc in range(8):
                for kq in range(_BLK // 128):
                    rowv = iota + (slot * (_BLK // 128) + kq) * ED
                    lanev = jnp.full((LANES,), cc, jnp.int32)
                    vs = []
                    for rl in range(ED):
                        vs.append(plsc.load_gather(tbuf, [rowv, lanev]))
                        lanev = lanev + 8
                    for rl in range(ED):
                        obuf[slot, kq * ED + rl, pl.ds(cc * ED, ED)] = vs[rl]
            out_copy(t, slot).start()

        @pl.when(valid(t + _NSLOT))
        def _():
            for cp in in_copies(t + _NSLOT, slot):
                cp.start()

    def loop_body(u, carry):
        for s in range(_NSLOT):
            phase(_NSLOT * u + s, s)
        return carry

    lax.fori_loop(0, _TPW // _NSLOT, loop_body, 0)

    for slot in range(_NSLOT):
        pltpu.make_async_copy(
            obuf.at[slot], tout.at[pl.ds(0, _OROWS), :], sems_o[slot]).wait()


def _make_sc_kernel(B: int):
    BPW = B // _NW
    NCHUNK = BPW // R

    mesh = plsc.VectorSubcoreMesh(core_axis_name="c", subcore_axis_name="s")

    @functools.partial(
        pl.kernel,
        mesh=mesh,
        out_type=jax.ShapeDtypeStruct((B, OUT_D), jnp.float32),
        scratch_types=[
            pltpu.VMEM((NS, R), jnp.int32),
            pltpu.VMEM((NS, R), jnp.int32),
            pltpu.VMEM((NS, R, ED), jnp.float32),
            pltpu.VMEM((NS, R), jnp.float32),
            pltpu.VMEM((ND, R), jnp.float32),
            pltpu.VMEM((LANES,), jnp.float32),
            pltpu.VMEM((R, OUT_D), jnp.float32),
            pltpu.SemaphoreType.DMA,
            pltpu.SemaphoreType.DMA,
        ],
        compiler_params=pltpu.CompilerParams(
            needs_layout_passes=False, use_tc_tiling_on_sc=False),
    )
    def sc_kernel(sp_hbm, dense_hbm, table_hbm, lin_hbm, w_hbm, out_hbm,
                  spbuf, idxbuf, gbuf, lbuf, dbuf, wbuf, obuf, sem_e, sem_l):
        nc = 2
        wid = lax.axis_index("s") * nc + lax.axis_index("c")
        wbase = wid * BPW
        iota = jnp.arange(LANES, dtype=jnp.int32)

        pltpu.sync_copy(w_hbm, wbuf)
        wv = wbuf[...]

        def chunk_body(c, carry):
            rbase = wbase + c * R

            pltpu.sync_copy(sp_hbm.at[:, pl.ds(rbase, R)], spbuf)
            pltpu.sync_copy(dense_hbm.at[:, pl.ds(rbase, R)], dbuf)

            for f in range(NS):
                for g in range(R // LANES):
                    sl = pl.ds(g * LANES, LANES)
                    idxbuf[f, sl] = spbuf[f, sl] + f * VOCAB

            emb_copies = []
            lin_copies = []
            for f in range(NS):
                emb_copies.append(
                    pltpu.async_copy(table_hbm.at[idxbuf.at[f]], gbuf.at[f],
                                     sem_e))
                lin_copies.append(
                    pltpu.async_copy(lin_hbm.at[idxbuf.at[f]], lbuf.at[f],
                                     sem_l))
            for cp in lin_copies:
                cp.wait()

            for g in range(R // LANES):
                sl = pl.ds(g * LANES, LANES)
                acc = jnp.zeros((LANES,), jnp.float32)
                for f in range(NS):
                    acc = acc + lbuf[f, sl]
                for d in range(ND):
                    acc = acc + dbuf[d, sl] * wv[d]
                plsc.store_scatter(
                    obuf, [iota + g * LANES,
                           jnp.full((LANES,), OUT_D - 1, jnp.int32)], acc)

            for cp in emb_copies:
                cp.wait()

            fvecs = [jnp.full((LANES,), f, jnp.int32) for f in range(NS)]

            def row_body(b, carry2):
                bvec = jnp.full((LANES,), b, jnp.int32)
                for f0 in range(0, NS, 13):
                    vs = [plsc.load_gather(gbuf, [fvecs[f], bvec, iota])
                          for f in range(f0, f0 + 13)]
                    for i, f in enumerate(range(f0, f0 + 13)):
                        obuf[b, pl.ds(f * ED, ED)] = vs[i]
                return carry2

            lax.fori_loop(0, R, row_body, 0)

            pltpu.sync_copy(obuf, out_hbm.at[pl.ds(rbase, R), :])
            return carry

        lax.fori_loop(0, NCHUNK, chunk_body, 0)

    return sc_kernel


def kernel(sparse_indices, dense, table, lin_table, lin_dense_w):
    B = sparse_indices.shape[0]
    sp_t = sparse_indices.astype(jnp.int32).T
    dense_t = dense.T
    table_rm = _sc_detile(table.T).reshape(_VPAD * 8, ED)
    lin_flat = lin_table.reshape(-1)
    w_pad = jnp.pad(lin_dense_w.reshape(-1), (0, LANES - ND))
    return _make_sc_kernel(B)(sp_t, dense_t, table_rm, lin_flat, w_pad)


# speedup vs baseline: 1.0149x; 1.0149x over previous
import functools

import jax
import jax.numpy as jnp
from jax import lax
from jax.experimental import pallas as pl
from jax.experimental.pallas import tpu as pltpu
from jax.experimental.pallas import tpu_sc as plsc

NS = 26
ND = 13
VOCAB = 100000
ED = 16
OUT_D = NS * ED + 1
LANES = 16
R = 128

_V = NS * VOCAB
_BLK = 1024
_NBLK = (_V + _BLK - 1) // _BLK
_OROWS = _BLK // 8
_VPAD = (_NBLK * _BLK) // 8
_NW = 32
_NSLOT = 3
_TPW = _NSLOT * ((_NBLK + _NSLOT * _NW - 1) // (_NSLOT * _NW))

_detile_mesh = plsc.VectorSubcoreMesh(core_axis_name="c", subcore_axis_name="s")


@functools.partial(
    pl.kernel,
    mesh=_detile_mesh,
    out_type=jax.ShapeDtypeStruct((_VPAD, 128), jnp.float32),
    scratch_types=[
        pltpu.VMEM((_NSLOT * (_BLK // 128) * ED, 128), jnp.float32),
        pltpu.VMEM((_NSLOT, _OROWS, 128), jnp.float32),
        pltpu.SemaphoreType.DMA,
        pltpu.SemaphoreType.DMA,
        pltpu.SemaphoreType.DMA,
        pltpu.SemaphoreType.DMA,
        pltpu.SemaphoreType.DMA,
        pltpu.SemaphoreType.DMA,
    ],
    compiler_params=pltpu.CompilerParams(needs_layout_passes=False),
)
def _sc_detile(tin, tout, tbuf, obuf, si0, si1, si2, so0, so1, so2):
    nc = 2
    wid = lax.axis_index("s") * nc + lax.axis_index("c")
    iota = jnp.arange(LANES, dtype=jnp.int32)
    sems_i = (si0, si1, si2)
    sems_o = (so0, so1, so2)
    last = _V + 64 - _BLK

    def base_of(t):
        return pl.multiple_of(jnp.minimum((t * _NW + wid) * _BLK, last), 128)

    def in_copies(t, slot):
        return [
            pltpu.make_async_copy(
                tin.at[:, pl.ds(base_of(t) + 128 * k, 128)],
                tbuf.at[pl.ds((slot * (_BLK // 128) + k) * ED, ED), :],
                sems_i[slot])
            for k in range(_BLK // 128)
        ]

    def out_copy(t, slot):
        return pltpu.make_async_copy(
            obuf.at[slot],
            tout.at[pl.ds(pl.multiple_of(base_of(t) // 8, 16), _OROWS), :],
            sems_o[slot])

    def valid(t):
        return (t * _NW + wid) * _BLK < _V

    for s in range(_NSLOT):
        for cp in in_copies(s, s):
            cp.start()

    def phase(t, slot):
        @pl.when(valid(t))
        def _():
            @pl.when(t >= _NSLOT)
            def _():
                out_copy(t - _NSLOT, slot).wait()
            for cp in in_copies(t, slot):
                cp.wait()
            for cc in range(8):
                for kq in range(_BLK // 128):
                    rowv = iota + (slot * (_BLK // 128) + kq) * ED
                    lanev = jnp.full((LANES,), cc, jnp.int32)
                    vs = []
                    for rl in range(ED):
                        vs.append(plsc.load_gather(tbuf, [rowv, lanev]))
                        lanev = lanev + 8
                    for rl in range(ED):
                        obuf[slot, kq * ED + rl, pl.ds(cc * ED, ED)] = vs[rl]
            out_copy(t, slot).start()

        @pl.when(valid(t + _NSLOT))
        def _():
            for cp in in_copies(t + _NSLOT, slot):
                cp.start()

    def loop_body(u, carry):
        for s in range(_NSLOT):
            phase(_NSLOT * u + s, s)
        return carry

    lax.fori_loop(0, _TPW // _NSLOT, loop_body, 0)

    for slot in range(_NSLOT):
        pltpu.make_async_copy(
            obuf.at[slot], tout.at[pl.ds(0, _OROWS), :], sems_o[slot]).wait()


def _make_sc_kernel(B: int):
    BPW = B // _NW
    NCHUNK = BPW // R

    mesh = plsc.VectorSubcoreMesh(core_axis_name="c", subcore_axis_name="s")

    @functools.partial(
        pl.kernel,
        mesh=mesh,
        out_type=jax.ShapeDtypeStruct((B * OUT_D,), jnp.float32),
        scratch_types=[
            pltpu.VMEM((NS, R), jnp.int32),
            pltpu.VMEM((NS, R), jnp.int32),
            pltpu.VMEM((NS, R, ED), jnp.float32),
            pltpu.VMEM((NS, R), jnp.float32),
            pltpu.VMEM((ND, R), jnp.float32),
            pltpu.VMEM((LANES,), jnp.float32),
            pltpu.VMEM((R * OUT_D,), jnp.float32),
            pltpu.SemaphoreType.DMA,
            pltpu.SemaphoreType.DMA,
        ],
        compiler_params=pltpu.CompilerParams(
            needs_layout_passes=False, use_tc_tiling_on_sc=False),
    )
    def sc_kernel(sp_hbm, dense_hbm, table_hbm, lin_hbm, w_hbm, out_hbm,
                  spbuf, idxbuf, gbuf, lbuf, dbuf, wbuf, obuf, sem_e, sem_l):
        nc = 2
        wid = lax.axis_index("s") * nc + lax.axis_index("c")
        wbase = wid * BPW
        iota = jnp.arange(LANES, dtype=jnp.int32)

        pltpu.sync_copy(w_hbm, wbuf)
        wv = wbuf[...]

        def chunk_body(c, carry):
            rbase = wbase + c * R

            pltpu.sync_copy(sp_hbm.at[:, pl.ds(rbase, R)], spbuf)
            pltpu.sync_copy(dense_hbm.at[:, pl.ds(rbase, R)], dbuf)

            for f in range(NS):
                for g in range(R // LANES):
                    sl = pl.ds(g * LANES, LANES)
                    idxbuf[f, sl] = spbuf[f, sl] + f * VOCAB

            emb_copies = []
            lin_copies = []
            for f in range(NS):
                emb_copies.append(
                    pltpu.async_copy(table_hbm.at[idxbuf.at[f]], gbuf.at[f],
                                     sem_e))
                lin_copies.append(
                    pltpu.async_copy(lin_hbm.at[idxbuf.at[f]], lbuf.at[f],
                                     sem_l))
            for cp in lin_copies:
                cp.wait()

            for g in range(R // LANES):
                sl = pl.ds(g * LANES, LANES)
                acc = jnp.zeros((LANES,), jnp.float32)
                for f in range(NS):
                    acc = acc + lbuf[f, sl]
                for d in range(ND):
                    acc = acc + dbuf[d, sl] * wv[d]
                plsc.store_scatter(
                    obuf, [iota * OUT_D + (g * LANES * OUT_D + OUT_D - 1)], acc)

            for cp in emb_copies:
                cp.wait()

            fvecs = [jnp.full((LANES,), f, jnp.int32) for f in range(NS)]

            def row_body(b, carry2):
                bvec = jnp.full((LANES,), b, jnp.int32)
                for f0 in range(0, NS, 13):
                    vs = [plsc.load_gather(gbuf, [fvecs[f], bvec, iota])
                          for f in range(f0, f0 + 13)]
                    for i, f in enumerate(range(f0, f0 + 13)):
                        obuf[pl.ds(b * OUT_D + f * ED, ED)] = vs[i]
                return carry2

            lax.fori_loop(0, R, row_body, 0)

            pltpu.sync_copy(obuf, out_hbm.at[pl.ds(rbase * OUT_D, R * OUT_D)])
            return carry

        lax.fori_loop(0, NCHUNK, chunk_body, 0)

    return sc_kernel


def kernel(sparse_indices, dense, table, lin_table, lin_dense_w):
    B = sparse_indices.shape[0]
    sp_t = sparse_indices.astype(jnp.int32).T
    dense_t = dense.T
    table_rm = _sc_detile(table.T).reshape(_VPAD * 8, ED)
    lin_flat = lin_table.reshape(-1)
    w_pad = jnp.pad(lin_dense_w.reshape(-1), (0, LANES - ND))
    out_flat = _make_sc_kernel(B)(sp_t, dense_t, table_rm, lin_flat, w_pad)
    return out_flat.reshape(B, OUT_D)


# speedup vs baseline: 1.0186x; 1.0036x over previous
import functools

import jax
import jax.numpy as jnp
from jax import lax
from jax.experimental import pallas as pl
from jax.experimental.pallas import tpu as pltpu
from jax.experimental.pallas import tpu_sc as plsc

NS = 26
ND = 13
VOCAB = 100000
ED = 16
OUT_D = NS * ED + 1
LANES = 16
R = 128

_V = NS * VOCAB
_BLK = 1024
_NBLK = (_V + _BLK - 1) // _BLK
_OROWS = _BLK // 8
_VPAD = (_NBLK * _BLK) // 8
_NW = 32
_NSLOT = 3
_TPW = _NSLOT * ((_NBLK + _NSLOT * _NW - 1) // (_NSLOT * _NW))

_detile_mesh = plsc.VectorSubcoreMesh(core_axis_name="c", subcore_axis_name="s")


@functools.partial(
    pl.kernel,
    mesh=_detile_mesh,
    out_type=jax.ShapeDtypeStruct((_VPAD, 128), jnp.float32),
    scratch_types=[
        pltpu.VMEM((_NSLOT * (_BLK // 128) * ED, 128), jnp.float32),
        pltpu.VMEM((_NSLOT, _OROWS, 128), jnp.float32),
        pltpu.SemaphoreType.DMA,
        pltpu.SemaphoreType.DMA,
        pltpu.SemaphoreType.DMA,
        pltpu.SemaphoreType.DMA,
        pltpu.SemaphoreType.DMA,
        pltpu.SemaphoreType.DMA,
    ],
    compiler_params=pltpu.CompilerParams(needs_layout_passes=False),
)
def _sc_detile(tin, tout, tbuf, obuf, si0, si1, si2, so0, so1, so2):
    nc = 2
    wid = lax.axis_index("s") * nc + lax.axis_index("c")
    iota = jnp.arange(LANES, dtype=jnp.int32)
    sems_i = (si0, si1, si2)
    sems_o = (so0, so1, so2)
    last = _V + 64 - _BLK

    def base_of(t):
        return pl.multiple_of(jnp.minimum((t * _NW + wid) * _BLK, last), 128)

    def in_copies(t, slot):
        return [
            pltpu.make_async_copy(
                tin.at[:, pl.ds(base_of(t) + 128 * k, 128)],
                tbuf.at[pl.ds((slot * (_BLK // 128) + k) * ED, ED), :],
                sems_i[slot])
            for k in range(_BLK // 128)
        ]

    def out_copy(t, slot):
        return pltpu.make_async_copy(
            obuf.at[slot],
            tout.at[pl.ds(pl.multiple_of(base_of(t) // 8, 16), _OROWS), :],
            sems_o[slot])

    def valid(t):
        return (t * _NW + wid) * _BLK < _V

    for s in range(_NSLOT):
        for cp in in_copies(s, s):
            cp.start()

    def phase(t, slot):
        @pl.when(valid(t))
        def _():
            @pl.when(t >= _NSLOT)
            def _():
                out_copy(t - _NSLOT, slot).wait()
            pltpu.make_async_copy(
                tout.at[pl.ds(0, (_BLK // 128) * ED), :],
                tbuf.at[pl.ds(slot * (_BLK // 128) * ED,
                              (_BLK // 128) * ED), :],
                sems_i[slot]).wait()
            for cc in range(8):
                for kq in range(_BLK // 128):
                    rowv = iota + (slot * (_BLK // 128) + kq) * ED
                    lanev = jnp.full((LANES,), cc, jnp.int32)
                    vs = []
                    for rl in range(ED):
                        vs.append(plsc.load_gather(tbuf, [rowv, lanev]))
                        lanev = lanev + 8
                    for rl in range(ED):
                        obuf[slot, kq * ED + rl, pl.ds(cc * ED, ED)] = vs[rl]
            out_copy(t, slot).start()

        @pl.when(valid(t + _NSLOT))
        def _():
            for cp in in_copies(t + _NSLOT, slot):
                cp.start()

    def loop_body(u, carry):
        for s in range(_NSLOT):
            phase(_NSLOT * u + s, s)
        return carry

    lax.fori_loop(0, _TPW // _NSLOT, loop_body, 0)

    for slot in range(_NSLOT):
        pltpu.make_async_copy(
            obuf.at[slot], tout.at[pl.ds(0, _OROWS), :], sems_o[slot]).wait()


def _make_sc_kernel(B: int):
    BPW = B // _NW
    NCHUNK = BPW // R

    mesh = plsc.VectorSubcoreMesh(core_axis_name="c", subcore_axis_name="s")

    @functools.partial(
        pl.kernel,
        mesh=mesh,
        out_type=jax.ShapeDtypeStruct((B * OUT_D,), jnp.float32),
        scratch_types=[
            pltpu.VMEM((NS, R), jnp.int32),
            pltpu.VMEM((NS, R), jnp.int32),
            pltpu.VMEM((NS, R, ED), jnp.float32),
            pltpu.VMEM((NS, R), jnp.float32),
            pltpu.VMEM((ND, R), jnp.float32),
            pltpu.VMEM((LANES,), jnp.float32),
            pltpu.VMEM((R * OUT_D,), jnp.float32),
            pltpu.SemaphoreType.DMA,
            pltpu.SemaphoreType.DMA,
        ],
        compiler_params=pltpu.CompilerParams(
            needs_layout_passes=False, use_tc_tiling_on_sc=False),
    )
    def sc_kernel(sp_hbm, dense_hbm, table_hbm, lin_hbm, w_hbm, out_hbm,
                  spbuf, idxbuf, gbuf, lbuf, dbuf, wbuf, obuf, sem_e, sem_l):
        nc = 2
        wid = lax.axis_index("s") * nc + lax.axis_index("c")
        wbase = wid * BPW
        iota = jnp.arange(LANES, dtype=jnp.int32)

        pltpu.sync_copy(w_hbm, wbuf)
        wv = wbuf[...]

        def chunk_body(c, carry):
            rbase = wbase + c * R

            pltpu.sync_copy(sp_hbm.at[:, pl.ds(rbase, R)], spbuf)
            pltpu.sync_copy(dense_hbm.at[:, pl.ds(rbase, R)], dbuf)

            for f in range(NS):
                for g in range(R // LANES):
                    sl = pl.ds(g * LANES, LANES)
                    idxbuf[f, sl] = spbuf[f, sl] + f * VOCAB

            emb_copies = []
            lin_copies = []
            for f in range(NS):
                emb_copies.append(
                    pltpu.async_copy(table_hbm.at[idxbuf.at[f]], gbuf.at[f],
                                     sem_e))
                lin_copies.append(
                    pltpu.async_copy(lin_hbm.at[idxbuf.at[f]], lbuf.at[f],
                                     sem_l))
            for cp in lin_copies:
                cp.wait()

            for g in range(R // LANES):
                sl = pl.ds(g * LANES, LANES)
                acc = jnp.zeros((LANES,), jnp.float32)
                for f in range(NS):
                    acc = acc + lbuf[f, sl]
                for d in range(ND):
                    acc = acc + dbuf[d, sl] * wv[d]
                plsc.store_scatter(
                    obuf, [iota * OUT_D + (g * LANES * OUT_D + OUT_D - 1)], acc)

            for cp in emb_copies:
                cp.wait()

            fvecs = [jnp.full((LANES,), f, jnp.int32) for f in range(NS)]

            def row_body(b, carry2):
                bvec = jnp.full((LANES,), b, jnp.int32)
                for f0 in range(0, NS, 13):
                    vs = [plsc.load_gather(gbuf, [fvecs[f], bvec, iota])
                          for f in range(f0, f0 + 13)]
                    for i, f in enumerate(range(f0, f0 + 13)):
                        obuf[pl.ds(b * OUT_D + f * ED, ED)] = vs[i]
                return carry2

            lax.fori_loop(0, R, row_body, 0)

            pltpu.sync_copy(obuf, out_hbm.at[pl.ds(rbase * OUT_D, R * OUT_D)])
            return carry

        lax.fori_loop(0, NCHUNK, chunk_body, 0)

    return sc_kernel


def kernel(sparse_indices, dense, table, lin_table, lin_dense_w):
    B = sparse_indices.shape[0]
    sp_t = sparse_indices.astype(jnp.int32).T
    dense_t = dense.T
    table_rm = _sc_detile(table.T).reshape(_VPAD * 8, ED)
    lin_flat = lin_table.reshape(-1)
    w_pad = jnp.pad(lin_dense_w.reshape(-1), (0, LANES - ND))
    out_flat = _make_sc_kernel(B)(sp_t, dense_t, table_rm, lin_flat, w_pad)
    return out_flat.reshape(B, OUT_D)
